# baseline (speedup 1.0000x reference)
import functools

import jax
import jax.numpy as jnp
from jax import lax
from jax.experimental import pallas as pl
from jax.experimental.pallas import tpu as pltpu
from jax.experimental.pallas import tpu_sc as plsc

B, S, H = 1, 2048, 768
E, K = 8, 2
INTER = 3072
LIMIT = 7.0
ALPHA = 1.702

N_FLAT = S * K
BM = 256
BN = 512
CB = INTER // BN
NB = N_FLAT // BM + E
N_PAD = NB * BM

NUM_WORKERS = 32
GCHUNK = 64


def _router_body(hs_ref, rw_ref, rb_ref, idx_ref, w_ref):
    logits = jnp.dot(hs_ref[...], rw_ref[...],
                     preferred_element_type=jnp.float32) + rb_ref[...]
    m1 = jnp.max(logits, axis=1)
    a1 = jnp.argmax(logits, axis=1).astype(jnp.int32)
    col = lax.broadcasted_iota(jnp.int32, (S, E), 1)
    masked = jnp.where(col == a1[:, None], -jnp.inf, logits)
    m2 = jnp.max(masked, axis=1)
    a2 = jnp.argmax(masked, axis=1).astype(jnp.int32)
    w1 = 1.0 / (1.0 + jnp.exp(m2 - m1))
    w2 = 1.0 - w1
    idx_ref[...] = jnp.where(col == 0, a1[:, None],
                             jnp.where(col == 1, a2[:, None], 0))
    w_ref[...] = jnp.where(col == 0, w1[:, None],
                           jnp.where(col == 1, w2[:, None], 0.0))


def _router(hs2d, router_weight, router_bias):
    return pl.pallas_call(
        _router_body,
        out_shape=(jax.ShapeDtypeStruct((S, E), jnp.int32),
                   jax.ShapeDtypeStruct((S, E), jnp.float32)),
    )(hs2d, router_weight, router_bias.reshape(1, E))


def _dispatch_body(hs_hbm, tok_hbm, out_hbm, idx_v, rows_v, sem):
    wid = lax.axis_index("s") * 2 + lax.axis_index("c")
    base = wid * (N_PAD // NUM_WORKERS)
    for c in range(N_PAD // NUM_WORKERS // GCHUNK):
        off = base + c * GCHUNK
        pltpu.sync_copy(tok_hbm.at[pl.ds(off, GCHUNK)], idx_v)
        pltpu.async_copy(hs_hbm.at[idx_v], rows_v, sem).wait()
        pltpu.sync_copy(rows_v, out_hbm.at[pl.ds(off, GCHUNK)])


@functools.cache
def _make_dispatch():
    return functools.partial(
        pl.kernel,
        mesh=plsc.VectorSubcoreMesh(core_axis_name="c", subcore_axis_name="s"),
        out_type=jax.ShapeDtypeStruct((N_PAD, H), jnp.float32),
        scratch_types=[
            pltpu.VMEM((GCHUNK,), jnp.int32),
            pltpu.VMEM((GCHUNK, H), jnp.float32),
            pltpu.SemaphoreType.DMA,
        ],
    )(_dispatch_body)


def _dispatch(hs2d, tok):
    return _make_dispatch()(hs2d, tok)


def _gateup_body(eob_ref, x_ref, wg_ref, wu_ref, bg_ref, bu_ref, act_ref):
    x = x_ref[...]
    gate = jnp.dot(x, wg_ref[0], preferred_element_type=jnp.float32) + bg_ref[0]
    up = jnp.dot(x, wu_ref[0], preferred_element_type=jnp.float32) + bu_ref[0]
    gate = jnp.minimum(gate, LIMIT)
    up = jnp.clip(up, -LIMIT, LIMIT)
    glu = gate * (1.0 / (1.0 + jnp.exp(-ALPHA * gate)))
    act_ref[...] = (up + 1.0) * glu


def _gateup(eob, x_sorted, gate_up_proj, gate_up_proj_bias):
    grid = (CB, NB)
    return pl.pallas_call(
        _gateup_body,
        grid_spec=pltpu.PrefetchScalarGridSpec(
            num_scalar_prefetch=1,
            grid=grid,
            in_specs=[
                pl.BlockSpec((BM, H), lambda cb, rb, eob: (rb, 0)),
                pl.BlockSpec((1, H, BN), lambda cb, rb, eob: (eob[rb], 0, cb)),
                pl.BlockSpec((1, H, BN), lambda cb, rb, eob: (eob[rb], 0, CB + cb)),
                pl.BlockSpec((1, 1, BN), lambda cb, rb, eob: (eob[rb], 0, cb)),
                pl.BlockSpec((1, 1, BN), lambda cb, rb, eob: (eob[rb], 0, CB + cb)),
            ],
            out_specs=pl.BlockSpec((BM, BN), lambda cb, rb, eob: (rb, cb)),
        ),
        out_shape=jax.ShapeDtypeStruct((N_PAD, INTER), jnp.float32),
    )(eob, x_sorted, gate_up_proj, gate_up_proj,
      gate_up_proj_bias.reshape(E, 1, 2 * INTER),
      gate_up_proj_bias.reshape(E, 1, 2 * INTER))


def _down_body(eob_ref, act_ref, wd_ref, bd_ref, w_ref, out_ref):
    y = jnp.dot(act_ref[...], wd_ref[0],
                preferred_element_type=jnp.float32) + bd_ref[0]
    out_ref[...] = y * w_ref[...]


def _down(eob, act, down_proj, down_proj_bias, w_rows):
    grid = (NB,)
    return pl.pallas_call(
        _down_body,
        grid_spec=pltpu.PrefetchScalarGridSpec(
            num_scalar_prefetch=1,
            grid=grid,
            in_specs=[
                pl.BlockSpec((BM, INTER), lambda rb, eob: (rb, 0)),
                pl.BlockSpec((1, INTER, H), lambda rb, eob: (eob[rb], 0, 0)),
                pl.BlockSpec((1, 1, H), lambda rb, eob: (eob[rb], 0, 0)),
                pl.BlockSpec((BM, 1), lambda rb, eob: (rb, 0)),
            ],
            out_specs=pl.BlockSpec((BM, H), lambda rb, eob: (rb, 0)),
        ),
        out_shape=jax.ShapeDtypeStruct((N_PAD, H), jnp.float32),
    )(eob, act, down_proj, down_proj_bias.reshape(E, 1, H), w_rows)


TOK_PER_W = S // NUM_WORKERS


def _combine_body(rows_hbm, p0_hbm, p1_hbm, out_hbm, i0_v, i1_v, a_v, b_v, sem):
    wid = lax.axis_index("s") * 2 + lax.axis_index("c")
    base = wid * TOK_PER_W
    pltpu.sync_copy(p0_hbm.at[pl.ds(base, TOK_PER_W)], i0_v)
    pltpu.sync_copy(p1_hbm.at[pl.ds(base, TOK_PER_W)], i1_v)
    pltpu.async_copy(rows_hbm.at[i0_v], a_v, sem).wait()
    pltpu.async_copy(rows_hbm.at[i1_v], b_v, sem).wait()

    def add_row(i, carry):
        for j in range(H // 16):
            sl = pl.ds(j * 16, 16)
            a_v[i, sl] += b_v[i, sl]
        return carry

    lax.fori_loop(0, TOK_PER_W, add_row, 0)
    pltpu.sync_copy(a_v, out_hbm.at[pl.ds(base, TOK_PER_W)])


@functools.cache
def _make_combine():
    return functools.partial(
        pl.kernel,
        mesh=plsc.VectorSubcoreMesh(core_axis_name="c", subcore_axis_name="s"),
        out_type=jax.ShapeDtypeStruct((S, H), jnp.float32),
        scratch_types=[
            pltpu.VMEM((TOK_PER_W,), jnp.int32),
            pltpu.VMEM((TOK_PER_W,), jnp.int32),
            pltpu.VMEM((TOK_PER_W, H), jnp.float32),
            pltpu.VMEM((TOK_PER_W, H), jnp.float32),
            pltpu.SemaphoreType.DMA,
        ],
    )(_combine_body)


def _combine(rows, p0, p1):
    return _make_combine()(rows, p0, p1)


def kernel(hidden_states, router_weight, router_bias, gate_up_proj,
           gate_up_proj_bias, down_proj, down_proj_bias):
    hs2d = hidden_states.reshape(S, H)

    idx_out, w_out = _router(hs2d, router_weight, router_bias)
    top_idx = idx_out[:, :K]
    w_flat = w_out[:, :K].reshape(-1)
    e_flat = top_idx.reshape(-1)

    order = jnp.argsort(e_flat, stable=True).astype(jnp.int32)
    g = jnp.bincount(e_flat, length=E)
    nb = (g + BM - 1) // BM
    cum_nb = jnp.cumsum(nb)
    b_ids = jnp.arange(NB, dtype=jnp.int32)
    eob = jnp.minimum(
        jnp.sum(b_ids[:, None] >= cum_nb[None, :], axis=1), E - 1
    ).astype(jnp.int32)
    po = jnp.concatenate([jnp.zeros(1, jnp.int32),
                          jnp.cumsum(nb * BM)[:-1].astype(jnp.int32)])
    raw_off = jnp.concatenate([jnp.zeros(1, jnp.int32),
                               jnp.cumsum(g)[:-1].astype(jnp.int32)])

    r = jnp.arange(N_PAD, dtype=jnp.int32)
    e_r = eob[r // BM]
    l = r - po[e_r]
    valid = (l >= 0) & (l < g[e_r])
    j = jnp.clip(raw_off[e_r] + l, 0, N_FLAT - 1)
    src = jnp.where(valid, order[j], 0)
    tok = (src // K).astype(jnp.int32)
    w_rows = jnp.where(valid, w_flat[src], 0.0).reshape(N_PAD, 1)

    pos = jnp.zeros((N_FLAT,), jnp.int32).at[
        jnp.where(valid, src, N_FLAT)].set(r, mode="drop")
    pos2 = pos.reshape(S, K)
    p0 = pos2[:, 0].astype(jnp.int32)
    p1 = pos2[:, 1].astype(jnp.int32)

    x_sorted = _dispatch(hs2d, tok)
    act = _gateup(eob, x_sorted, gate_up_proj, gate_up_proj_bias)
    rows = _down(eob, act, down_proj, down_proj_bias, w_rows)
    out = _combine(rows, p0, p1)
    return out.reshape(B, S, H)


# speedup vs baseline: 1.0197x; 1.0197x over previous
import functools

import jax
import jax.numpy as jnp
from jax import lax
from jax.experimental import pallas as pl
from jax.experimental.pallas import tpu as pltpu
from jax.experimental.pallas import tpu_sc as plsc

B, S, H = 1, 2048, 768
E, K = 8, 2
INTER = 3072
LIMIT = 7.0
ALPHA = 1.702

N_FLAT = S * K
BM = 256
BN = 512
CB = INTER // BN
NB = N_FLAT // BM + E
N_PAD = NB * BM

NUM_WORKERS = 32
GCHUNK = 64


def _router_body(hs_ref, rw_ref, rb_ref, idx_ref, w_ref):
    logits = jnp.dot(hs_ref[...], rw_ref[...],
                     preferred_element_type=jnp.float32) + rb_ref[...]
    m1 = jnp.max(logits, axis=1)
    a1 = jnp.argmax(logits, axis=1).astype(jnp.int32)
    col = lax.broadcasted_iota(jnp.int32, (S, E), 1)
    masked = jnp.where(col == a1[:, None], -jnp.inf, logits)
    m2 = jnp.max(masked, axis=1)
    a2 = jnp.argmax(masked, axis=1).astype(jnp.int32)
    w1 = 1.0 / (1.0 + jnp.exp(m2 - m1))
    w2 = 1.0 - w1
    idx_ref[...] = jnp.where(col == 0, a1[:, None],
                             jnp.where(col == 1, a2[:, None], 0))
    w_ref[...] = jnp.where(col == 0, w1[:, None],
                           jnp.where(col == 1, w2[:, None], 0.0))


def _router(hs2d, router_weight, router_bias):
    return pl.pallas_call(
        _router_body,
        out_shape=(jax.ShapeDtypeStruct((S, E), jnp.int32),
                   jax.ShapeDtypeStruct((S, E), jnp.float32)),
    )(hs2d, router_weight, router_bias.reshape(1, E))


ROWS_PER_W = N_PAD // NUM_WORKERS
NCHUNK = ROWS_PER_W // GCHUNK


def _dispatch_body(hs_hbm, tok_hbm, out_hbm, idx_v, rows0, rows1, gsem, ssem):
    wid = lax.axis_index("s") * 2 + lax.axis_index("c")
    base = wid * ROWS_PER_W
    bufs = (rows0, rows1)
    pltpu.sync_copy(tok_hbm.at[pl.ds(base, ROWS_PER_W)], idx_v)
    g0 = pltpu.async_copy(hs_hbm.at[idx_v.at[pl.ds(0, GCHUNK)]], rows0, gsem)
    pending = g0
    stores = []
    for c in range(NCHUNK):
        if c + 1 < NCHUNK:
            if c >= 1:
                stores[c - 1].wait()
            nxt = pltpu.async_copy(
                hs_hbm.at[idx_v.at[pl.ds((c + 1) * GCHUNK, GCHUNK)]],
                bufs[(c + 1) % 2], gsem)
        pending.wait()
        st = pltpu.async_copy(
            bufs[c % 2], out_hbm.at[pl.ds(base + c * GCHUNK, GCHUNK)], ssem)
        stores.append(st)
        if c + 1 < NCHUNK:
            pending = nxt
    for st in stores[-2:]:
        st.wait()


@functools.cache
def _make_dispatch():
    return functools.partial(
        pl.kernel,
        mesh=plsc.VectorSubcoreMesh(core_axis_name="c", subcore_axis_name="s"),
        out_type=jax.ShapeDtypeStruct((N_PAD, H), jnp.float32),
        scratch_types=[
            pltpu.VMEM((ROWS_PER_W,), jnp.int32),
            pltpu.VMEM((GCHUNK, H), jnp.float32),
            pltpu.VMEM((GCHUNK, H), jnp.float32),
            pltpu.SemaphoreType.DMA,
            pltpu.SemaphoreType.DMA,
        ],
    )(_dispatch_body)


def _dispatch(hs2d, tok):
    return _make_dispatch()(hs2d, tok)


def _gateup_body(eob_ref, x_ref, wg_ref, wu_ref, bg_ref, bu_ref, act_ref):
    x = x_ref[...].astype(jnp.bfloat16)
    gate = jnp.dot(x, wg_ref[0], preferred_element_type=jnp.float32) + bg_ref[0]
    up = jnp.dot(x, wu_ref[0], preferred_element_type=jnp.float32) + bu_ref[0]
    gate = jnp.minimum(gate, LIMIT)
    up = jnp.clip(up, -LIMIT, LIMIT)
    glu = gate * (1.0 / (1.0 + jnp.exp(-ALPHA * gate)))
    act_ref[...] = ((up + 1.0) * glu).astype(jnp.bfloat16)


def _gateup(eob, x_sorted, gate_up_proj, gate_up_proj_bias):
    grid = (CB, NB)
    return pl.pallas_call(
        _gateup_body,
        grid_spec=pltpu.PrefetchScalarGridSpec(
            num_scalar_prefetch=1,
            grid=grid,
            in_specs=[
                pl.BlockSpec((BM, H), lambda cb, rb, eob: (rb, 0)),
                pl.BlockSpec((1, H, BN), lambda cb, rb, eob: (eob[rb], 0, cb)),
                pl.BlockSpec((1, H, BN), lambda cb, rb, eob: (eob[rb], 0, CB + cb)),
                pl.BlockSpec((1, 1, BN), lambda cb, rb, eob: (eob[rb], 0, cb)),
                pl.BlockSpec((1, 1, BN), lambda cb, rb, eob: (eob[rb], 0, CB + cb)),
            ],
            out_specs=pl.BlockSpec((BM, BN), lambda cb, rb, eob: (rb, cb)),
        ),
        out_shape=jax.ShapeDtypeStruct((N_PAD, INTER), jnp.bfloat16),
    )(eob, x_sorted, gate_up_proj, gate_up_proj,
      gate_up_proj_bias.reshape(E, 1, 2 * INTER),
      gate_up_proj_bias.reshape(E, 1, 2 * INTER))


def _down_body(eob_ref, act_ref, wd_ref, bd_ref, w_ref, out_ref):
    y = jnp.dot(act_ref[...], wd_ref[0],
                preferred_element_type=jnp.float32) + bd_ref[0]
    out_ref[...] = y * w_ref[...]


def _down(eob, act, down_proj, down_proj_bias, w_rows):
    grid = (NB,)
    return pl.pallas_call(
        _down_body,
        grid_spec=pltpu.PrefetchScalarGridSpec(
            num_scalar_prefetch=1,
            grid=grid,
            in_specs=[
                pl.BlockSpec((BM, INTER), lambda rb, eob: (rb, 0)),
                pl.BlockSpec((1, INTER, H), lambda rb, eob: (eob[rb], 0, 0)),
                pl.BlockSpec((1, 1, H), lambda rb, eob: (eob[rb], 0, 0)),
                pl.BlockSpec((BM, 1), lambda rb, eob: (rb, 0)),
            ],
            out_specs=pl.BlockSpec((BM, H), lambda rb, eob: (rb, 0)),
        ),
        out_shape=jax.ShapeDtypeStruct((N_PAD, H), jnp.float32),
    )(eob, act, down_proj, down_proj_bias.reshape(E, 1, H), w_rows)


TOK_PER_W = S // NUM_WORKERS


def _combine_body(rows_hbm, p0_hbm, p1_hbm, out_hbm, i0_v, i1_v, a_v, b_v, sem):
    wid = lax.axis_index("s") * 2 + lax.axis_index("c")
    base = wid * TOK_PER_W
    pltpu.sync_copy(p0_hbm.at[pl.ds(base, TOK_PER_W)], i0_v)
    pltpu.sync_copy(p1_hbm.at[pl.ds(base, TOK_PER_W)], i1_v)
    pltpu.async_copy(rows_hbm.at[i0_v], a_v, sem).wait()
    pltpu.async_copy(rows_hbm.at[i1_v], b_v, sem).wait()

    def add_row(i, carry):
        for j in range(H // 16):
            sl = pl.ds(j * 16, 16)
            a_v[i, sl] += b_v[i, sl]
        return carry

    lax.fori_loop(0, TOK_PER_W, add_row, 0)
    pltpu.sync_copy(a_v, out_hbm.at[pl.ds(base, TOK_PER_W)])


@functools.cache
def _make_combine():
    return functools.partial(
        pl.kernel,
        mesh=plsc.VectorSubcoreMesh(core_axis_name="c", subcore_axis_name="s"),
        out_type=jax.ShapeDtypeStruct((S, H), jnp.float32),
        scratch_types=[
            pltpu.VMEM((TOK_PER_W,), jnp.int32),
            pltpu.VMEM((TOK_PER_W,), jnp.int32),
            pltpu.VMEM((TOK_PER_W, H), jnp.float32),
            pltpu.VMEM((TOK_PER_W, H), jnp.float32),
            pltpu.SemaphoreType.DMA,
        ],
    )(_combine_body)


def _combine(rows, p0, p1):
    return _make_combine()(rows, p0, p1)


def kernel(hidden_states, router_weight, router_bias, gate_up_proj,
           gate_up_proj_bias, down_proj, down_proj_bias):
    hs2d = hidden_states.reshape(S, H)

    idx_out, w_out = _router(hs2d, router_weight, router_bias)
    top_idx = idx_out[:, :K]
    w_flat = w_out[:, :K].reshape(-1)
    e_flat = top_idx.reshape(-1)

    oh = (e_flat[:, None] == jnp.arange(E, dtype=jnp.int32)[None, :])
    csum = jnp.cumsum(oh.astype(jnp.int32), axis=0)
    g = csum[-1]
    rank = jnp.sum(jnp.where(oh, csum, 0), axis=1) - 1
    nb = (g + BM - 1) // BM
    cum_nb = jnp.cumsum(nb)
    b_ids = jnp.arange(NB, dtype=jnp.int32)
    eob = jnp.minimum(
        jnp.sum(b_ids[:, None] >= cum_nb[None, :], axis=1), E - 1
    ).astype(jnp.int32)
    po = jnp.concatenate([jnp.zeros(1, jnp.int32),
                          (jnp.cumsum(nb * BM)[:-1]).astype(jnp.int32)])

    pos = (jnp.sum(jnp.where(oh, po[None, :], 0), axis=1)
           + rank).astype(jnp.int32)
    pair = jnp.arange(N_FLAT, dtype=jnp.int32)
    tok = jnp.zeros((N_PAD,), jnp.int32).at[pos].set(pair // K)
    w_rows = jnp.zeros((N_PAD,), jnp.float32).at[pos].set(
        w_flat).reshape(N_PAD, 1)
    pos2 = pos.reshape(S, K)
    p0 = pos2[:, 0]
    p1 = pos2[:, 1]

    x_sorted = _dispatch(hs2d, tok)
    act = _gateup(eob, x_sorted, gate_up_proj.astype(jnp.bfloat16),
                  gate_up_proj_bias)
    rows = _down(eob, act, down_proj.astype(jnp.bfloat16), down_proj_bias,
                 w_rows)
    out = _combine(rows, p0, p1)
    return out.reshape(B, S, H)
